# baseline (speedup 1.0000x reference)
import functools

import jax
import jax.numpy as jnp
from jax import lax
from jax.experimental import pallas as pl
from jax.experimental.pallas import tpu as pltpu
from jax.experimental.pallas import tpu_sc as plsc

B = 16384
N_DENSE = 13
N_SPARSE = 26
VOCAB = 100000
EMBED = 16
ROWS = N_SPARSE * EMBED

NW = 32
RPW = ROWS // NW
CH = 4096
NCH = B // CH
TPC = CH // 16


def _sc_gather_t(embT, idxT):
    mesh = plsc.VectorSubcoreMesh(core_axis_name="c", subcore_axis_name="s")

    @functools.partial(
        pl.kernel,
        out_type=jax.ShapeDtypeStruct((ROWS, B), jnp.float32),
        mesh=mesh,
        scratch_types=[
            pltpu.VMEM((VOCAB,), jnp.float32),
            pltpu.VMEM((B,), jnp.int32),
            pltpu.VMEM((2 * CH,), jnp.float32),
            pltpu.SemaphoreType.DMA,
            pltpu.SemaphoreType.DMA,
            pltpu.SemaphoreType.DMA,
            pltpu.SemaphoreType.DMA,
        ],
        compiler_params=pltpu.CompilerParams(
            use_tc_tiling_on_sc=True, needs_layout_passes=False
        ),
    )
    def k(embT_hbm, idxT_hbm, out_hbm, rowb, idxb, outb, semr, semi, semo0, semo1):
        wid = lax.axis_index("s") * 2 + lax.axis_index("c")
        r0 = wid * RPW
        semo = (semo0, semo1)

        @pl.loop(0, RPW)
        def _row(i):
            r = r0 + i
            f = lax.shift_right_logical(r, 4)
            cpr = pltpu.async_copy(embT_hbm.at[r], rowb, semr)
            cpi = pltpu.async_copy(idxT_hbm.at[f], idxb, semi)
            cpi.wait()
            cpr.wait()
            for c in range(NCH):
                b = c & 1
                ob = outb.at[pl.ds(b * CH, CH)]
                oh = out_hbm.at[r, pl.ds(c * CH, CH)]

                def _drain(ob=ob, oh=oh, s=semo[b]):
                    pltpu.make_async_copy(ob, oh, s).wait()

                if c >= 2:
                    _drain()
                else:
                    pl.when(i > 0)(_drain)

                U = 8
                @pl.loop(0, TPC // U)
                def _vec(t, c=c, b=b):
                    bi = c * CH + t * (16 * U)
                    bo = b * CH + t * (16 * U)
                    ivs = [idxb[pl.ds(bi + k * 16, 16)] for k in range(U)]
                    gs = [plsc.load_gather(rowb, [iv]) for iv in ivs]
                    for k in range(U):
                        outb[pl.ds(bo + k * 16, 16)] = gs[k]

                pltpu.async_copy(ob, oh, semo[b])

        rl = r0 + RPW - 1
        for c in range(NCH - 2, NCH):
            b = c & 1
            pltpu.make_async_copy(
                outb.at[pl.ds(b * CH, CH)],
                out_hbm.at[rl, pl.ds(c * CH, CH)],
                semo[b],
            ).wait()

    return k(embT, idxT)


TB = 2048


def _mlp_body(xt_ref, e_ref, w1a_ref, w1b_ref, b1_ref, w2_ref, b2_ref,
              w3_ref, b3_ref, wws_ref, wod_ref, cz_ref, o_ref):
    dense_t = xt_ref[:N_DENSE, :]
    e = e_ref[:]
    h = jnp.dot(w1a_ref[:], dense_t, preferred_element_type=jnp.float32)
    h = h + jnp.dot(w1b_ref[:], e, preferred_element_type=jnp.float32)
    h = jnp.maximum(h + b1_ref[:], 0.0)
    h = jnp.dot(w2_ref[:], h, preferred_element_type=jnp.float32)
    h = jnp.maximum(h + b2_ref[:], 0.0)
    h = jnp.dot(w3_ref[:], h, preferred_element_type=jnp.float32)
    h = jnp.maximum(h + b3_ref[:], 0.0)
    z = jnp.dot(wod_ref[:], h, preferred_element_type=jnp.float32)
    z = z + jnp.dot(wws_ref[:], e, preferred_element_type=jnp.float32)
    o_ref[:] = jax.nn.sigmoid(z + cz_ref[0, 0])


def _tc_mlp(xt, e, W1aT, W1bT, b1c, W2T, b2c, W3T, b3c, WwsT, WodT, cz):
    grid = (B // TB,)
    full = lambda a: pl.BlockSpec(a.shape, lambda i: (0,) * a.ndim)
    return pl.pallas_call(
        _mlp_body,
        grid=grid,
        in_specs=[
            pl.BlockSpec((xt.shape[0], TB), lambda i: (0, i)),
            pl.BlockSpec((ROWS, TB), lambda i: (0, i)),
            full(W1aT), full(W1bT), full(b1c), full(W2T), full(b2c),
            full(W3T), full(b3c), full(WwsT), full(WodT), full(cz),
        ],
        out_specs=pl.BlockSpec((1, TB), lambda i: (0, i)),
        out_shape=jax.ShapeDtypeStruct((1, B), jnp.float32),
    )(xt, e, W1aT, W1bT, b1c, W2T, b2c, W3T, b3c, WwsT, WodT, cz)


def kernel(inputs, emb, W1, b1, W2, b2, W3, b3, Ww, bw, Wo, bo):
    embT = emb.transpose(0, 2, 1).reshape(ROWS, VOCAB)
    inputsT = inputs.T
    idxT = jnp.clip(inputsT[N_DENSE:].astype(jnp.int32), 0, VOCAB - 1)

    embTg = _sc_gather_t(embT, idxT)

    c_wide = Wo[-1, 0]
    W1aT = W1[:N_DENSE].T
    W1bT = W1[N_DENSE:].T
    W2T = W2.T
    W3T = W3.T
    WodT = Wo[:-1].T
    WwsT = (Ww * c_wide).T
    cz = (bw[0] * c_wide + bo[0]).reshape(1, 1)

    out_t = _tc_mlp(inputsT, embTg, W1aT, W1bT, b1.reshape(-1, 1),
                    W2T, b2.reshape(-1, 1), W3T, b3.reshape(-1, 1),
                    WwsT, WodT, cz)
    return out_t.T
